# baseline (speedup 1.0000x reference)
import numpy as np
import jax
import jax.numpy as jnp
from jax import lax
from jax.experimental import pallas as pl
from jax.experimental.pallas import tpu as pltpu
from jax.experimental.pallas import tpu_sc as plsc

F = 26
B = 26
V = 100000
D = 32
K = 4
P = B * (B - 1) // 2
L = 16

BPAD = 32
PPAD = 336
NCHUNK_FULL = P // L
TAIL = P - NCHUNK_FULL * L
NBUF = 4


def _body(xt_hbm, tab_hbm, wf_hbm, ridx_hbm, cidx_hbm, out_hbm,
          xt_v, w_v, r_v, c_v, e_v, obuf, stg, sems, osem):
    fid = lax.axis_index("s") * 2 + lax.axis_index("c")
    lanes = lax.iota(jnp.int32, L)

    pltpu.sync_copy(xt_hbm, xt_v)
    pltpu.sync_copy(wf_hbm, w_v)
    pltpu.sync_copy(ridx_hbm, r_v)
    pltpu.sync_copy(cidx_hbm, c_v)

    @pl.when(fid < F)
    def _():
        xbase = pl.multiple_of(fid * BPAD, BPAD)
        g_lo = xt_v[pl.ds(xbase, L)]
        g_hi = xt_v[pl.ds(xbase + L, L)]

        descs = [None] * NBUF
        gs = [None] * B

        def extract(b):
            g = gs[b]
            rm = g - (g // 8) * 8
            rsp = jnp.zeros((L,), jnp.int32) + rm
            s = stg[b % NBUF]
            lo = plsc.load_gather(s, [rsp, lanes])
            hi = plsc.load_gather(s, [rsp, lanes + L])
            e_v[pl.ds(b * D, L)] = lo
            e_v[pl.ds(b * D + L, L)] = hi

        for b in range(B):
            g = g_lo[b] if b < L else g_hi[b - L]
            gs[b] = g
            blk8 = pl.multiple_of((g // 8) * 8, 8)
            descs[b % NBUF] = pltpu.async_copy(
                tab_hbm.at[fid, pl.ds(blk8, 8), :], stg[b % NBUF],
                sems[b % NBUF])
            if b >= NBUF - 1:
                descs[(b - NBUF + 1) % NBUF].wait()
                extract(b - NBUF + 1)
        for b in range(B - NBUF + 1, B):
            descs[b % NBUF].wait()
            extract(b)

        iota_h = [lanes, lanes + L]

        def do_pair(p_off, r, c, s):
            rsp = jnp.zeros((L,), jnp.int32) + r * D
            csp = jnp.zeros((L,), jnp.int32) + c * D
            sv = jnp.zeros((L,), jnp.float32) + s
            for h in range(2):
                er = plsc.load_gather(e_v, [rsp + iota_h[h]])
                ec = plsc.load_gather(e_v, [csp + iota_h[h]])
                obuf[pl.ds(p_off + h * L, L)] = er * ec * sv

        def chunk_scores(r_vec, c_vec):
            score = None
            for k in range(K):
                wr = plsc.load_gather(w_v, [r_vec * K + k])
                wc = plsc.load_gather(w_v, [c_vec * K + k])
                prod = wr * wc
                score = prod if score is None else score + prod
            return score

        def chunk_body(ch, carry):
            cbase = pl.multiple_of(ch * L, L)
            r_vec = r_v[pl.ds(cbase, L)]
            c_vec = c_v[pl.ds(cbase, L)]
            s_vec = chunk_scores(r_vec, c_vec)
            pbase = pl.multiple_of(ch * (L * D), L * D)
            for j in range(L):
                do_pair(pbase + j * D, r_vec[j], c_vec[j], s_vec[j])
            return carry

        lax.fori_loop(0, NCHUNK_FULL, chunk_body, 0)

        r_vec = r_v[pl.ds(NCHUNK_FULL * L, L)]
        c_vec = c_v[pl.ds(NCHUNK_FULL * L, L)]
        s_vec = chunk_scores(r_vec, c_vec)
        for j in range(TAIL):
            do_pair((NCHUNK_FULL * L + j) * D, r_vec[j], c_vec[j], s_vec[j])

        pltpu.async_copy(obuf, out_hbm.at[pl.ds(fid * (P * D), P * D)],
                         osem).wait()


@jax.jit
def _ffm_sc(xt, tab, w_flat, ridx, cidx):
    mesh = plsc.VectorSubcoreMesh(core_axis_name="c", subcore_axis_name="s")
    return pl.kernel(
        _body,
        out_type=jax.ShapeDtypeStruct((F * P * D,), jnp.float32),
        mesh=mesh,
        compiler_params=pltpu.CompilerParams(needs_layout_passes=False),
        scratch_types=[
            pltpu.VMEM((F * BPAD,), jnp.int32),
            pltpu.VMEM((F * K,), jnp.float32),
            pltpu.VMEM((PPAD,), jnp.int32),
            pltpu.VMEM((PPAD,), jnp.int32),
            pltpu.VMEM((B * D,), jnp.float32),
            pltpu.VMEM((P * D,), jnp.float32),
            [pltpu.VMEM((8, D), jnp.float32) for _ in range(NBUF)],
            [pltpu.SemaphoreType.DMA for _ in range(NBUF)],
            pltpu.SemaphoreType.DMA,
        ],
    )(xt, tab, w_flat, ridx, cidx)


def kernel(x, tables, W):
    xt = jnp.pad(x.T.astype(jnp.int32), ((0, 0), (0, BPAD - B))).reshape(-1)
    w_flat = W.reshape(-1)
    r_np, c_np = np.triu_indices(B, k=1)
    ridx = jnp.pad(jnp.asarray(r_np, jnp.int32), (0, PPAD - P))
    cidx = jnp.pad(jnp.asarray(c_np, jnp.int32), (0, PPAD - P))
    return _ffm_sc(xt, tables, w_flat, ridx, cidx).reshape(F, P * D)


# speedup vs baseline: 21.4118x; 21.4118x over previous
import numpy as np
import jax
import jax.numpy as jnp
from jax import lax
from jax.experimental import pallas as pl
from jax.experimental.pallas import tpu as pltpu
from jax.experimental.pallas import tpu_sc as plsc

F = 26
B = 26
V = 100000
D = 32
K = 4
P = B * (B - 1) // 2
L = 16

PPAD = 336
NCHUNK = PPAD // L
NBUF = 4

_r0, _c0 = np.triu_indices(B, k=1)
_order = np.lexsort((_r0, _c0))
_RS = _r0[_order].tolist()
_CS = _c0[_order].tolist()
_POS = _order.tolist()
_RC_NP = np.zeros((PPAD,), np.int32)
_RC_NP[:P] = np.asarray(_RS, np.int32) * 32 + np.asarray(_CS, np.int32)
_CMAX = [_CS[min(ch * L + L - 1, P - 1)] for ch in range(NCHUNK)]


def _body(x_hbm, tab_hbm, wt_hbm, rc_hbm, out_hbm,
          x_v, w_v, rc_v, e_v, obuf, stg, sems, osem):
    fid = lax.axis_index("s") * 2 + lax.axis_index("c")
    lanes = lax.iota(jnp.int32, L)

    in_copies = [pltpu.async_copy(x_hbm, x_v, sems[0]),
                 pltpu.async_copy(wt_hbm, w_v, sems[1]),
                 pltpu.async_copy(rc_hbm, rc_v, sems[2])]
    for cp in in_copies:
        cp.wait()

    @pl.when(fid < F)
    def _():
        fsp = jnp.zeros((L,), jnp.int32) + fid
        g_lo = plsc.load_gather(x_v, [lanes, fsp])
        g_hi = plsc.load_gather(x_v, [lanes + L, fsp], mask=lanes < B - L)

        descs = [None] * NBUF
        state = {"issued": 0, "drained": 0}

        def issue_one():
            b = state["issued"]
            g = g_lo[b] if b < L else g_hi[b - L]
            vblk = pl.multiple_of((g // 128) * 128, 128)
            descs[b % NBUF] = pltpu.async_copy(
                tab_hbm.at[fid, :, pl.ds(vblk, 128)], stg[b % NBUF],
                sems[b % NBUF])
            state["issued"] = b + 1

        def drain_one():
            b = state["drained"]
            descs[b % NBUF].wait()
            g = g_lo[b] if b < L else g_hi[b - L]
            vm = g - (g // 128) * 128
            vsp = jnp.zeros((L,), jnp.int32) + vm
            s = stg[b % NBUF]
            e_v[pl.ds(b * D, L)] = plsc.load_gather(s, [lanes, vsp])
            e_v[pl.ds(b * D + L, L)] = plsc.load_gather(s, [lanes + L, vsp])
            state["drained"] = b + 1

        ksp = [jnp.zeros((L,), jnp.int32) + k for k in range(K)]

        def chunk_static(ch):
            rc = rc_v[pl.ds(ch * L, L)]
            r_vec = lax.shift_right_logical(rc, 5)
            c_vec = lax.bitwise_and(rc, 31)
            score = None
            for k in range(K):
                wr = plsc.load_gather(w_v, [ksp[k], r_vec])
                wc = plsc.load_gather(w_v, [ksp[k], c_vec])
                prod = wr * wc
                score = prod if score is None else score + prod
            ec_cache = {}
            for j in range(L):
                pg = ch * L + j
                if pg >= P:
                    break
                r, c, po = _RS[pg], _CS[pg], _POS[pg] * D
                sv = jnp.zeros((L,), jnp.float32) + score[j]
                for h in (0, L):
                    if (c, h) not in ec_cache:
                        ec_cache[(c, h)] = e_v[pl.ds(c * D + h, L)]
                    er = e_v[pl.ds(r * D + h, L)]
                    obuf[pl.ds(po + h, L)] = er * ec_cache[(c, h)] * sv

        while state["issued"] < NBUF:
            issue_one()
        for ch in range(NCHUNK):
            need = _CMAX[ch] + 1
            while state["drained"] < need:
                drain_one()
                if state["issued"] < B:
                    issue_one()
            chunk_static(ch)

        pltpu.async_copy(obuf, out_hbm.at[pl.ds(fid * (P * D), P * D)],
                         osem).wait()


@jax.jit
def _ffm_sc(x, tab_t, w_t, rc):
    mesh = plsc.VectorSubcoreMesh(core_axis_name="c", subcore_axis_name="s")
    return pl.kernel(
        _body,
        out_type=jax.ShapeDtypeStruct((F * P * D,), jnp.float32),
        mesh=mesh,
        compiler_params=pltpu.CompilerParams(needs_layout_passes=False),
        scratch_types=[
            pltpu.VMEM((B, F), jnp.int32),
            pltpu.VMEM((K, B), jnp.float32),
            pltpu.VMEM((PPAD,), jnp.int32),
            pltpu.VMEM((B * D,), jnp.float32),
            pltpu.VMEM((P * D,), jnp.float32),
            [pltpu.VMEM((D, 128), jnp.float32) for _ in range(NBUF)],
            [pltpu.SemaphoreType.DMA for _ in range(NBUF)],
            pltpu.SemaphoreType.DMA,
        ],
    )(x, tab_t, w_t, rc)


def kernel(x, tables, W):
    tab_t = jnp.swapaxes(tables, 1, 2)
    w_t = jnp.swapaxes(W, 0, 1)
    rc = jnp.asarray(_RC_NP)
    return _ffm_sc(x, tab_t, w_t, rc).reshape(F, P * D)


# speedup vs baseline: 22.4052x; 1.0464x over previous
import numpy as np
import jax
import jax.numpy as jnp
from jax import lax
from jax.experimental import pallas as pl
from jax.experimental.pallas import tpu as pltpu
from jax.experimental.pallas import tpu_sc as plsc

F = 26
B = 26
V = 100000
D = 32
K = 4
P = B * (B - 1) // 2
L = 16

PPAD = 336
NCHUNK = PPAD // L
NBUF = 8

_r0, _c0 = np.triu_indices(B, k=1)
_order = np.lexsort((_r0, _c0))
_RS = _r0[_order].tolist()
_CS = _c0[_order].tolist()
_POS = _order.tolist()
_RC_NP = np.zeros((PPAD,), np.int32)
_RC_NP[:P] = np.asarray(_RS, np.int32) * 32 + np.asarray(_CS, np.int32)
_CMAX = [_CS[min(ch * L + L - 1, P - 1)] for ch in range(NCHUNK)]


def _body(x_hbm, tab_hbm, wt_hbm, rc_hbm, out_hbm,
          x_v, w_v, rc_v, e_v, obuf, stg, sems, osem):
    fid = lax.axis_index("s") * 2 + lax.axis_index("c")
    lanes = lax.iota(jnp.int32, L)

    in_copies = [pltpu.async_copy(x_hbm, x_v, sems[0]),
                 pltpu.async_copy(wt_hbm, w_v, sems[1]),
                 pltpu.async_copy(rc_hbm, rc_v, sems[2])]
    for cp in in_copies:
        cp.wait()

    @pl.when(fid < F)
    def _():
        fsp = jnp.zeros((L,), jnp.int32) + fid
        g_lo = plsc.load_gather(x_v, [lanes, fsp])
        g_hi = plsc.load_gather(x_v, [lanes + L, fsp], mask=lanes < B - L)

        descs = [None] * NBUF
        state = {"issued": 0, "drained": 0}

        def issue_one():
            b = state["issued"]
            g = g_lo[b] if b < L else g_hi[b - L]
            vblk = pl.multiple_of((g // 128) * 128, 128)
            descs[b % NBUF] = pltpu.async_copy(
                tab_hbm.at[fid, :, pl.ds(vblk, 128)], stg[b % NBUF],
                sems[b % NBUF])
            state["issued"] = b + 1

        def drain_one():
            b = state["drained"]
            descs[b % NBUF].wait()
            g = g_lo[b] if b < L else g_hi[b - L]
            vm = g - (g // 128) * 128
            vsp = jnp.zeros((L,), jnp.int32) + vm
            s = stg[b % NBUF]
            e_v[pl.ds(b * D, L)] = plsc.load_gather(s, [lanes, vsp])
            e_v[pl.ds(b * D + L, L)] = plsc.load_gather(s, [lanes + L, vsp])
            state["drained"] = b + 1

        ksp = [jnp.zeros((L,), jnp.int32) + k for k in range(K)]

        def chunk_static(ch):
            rc = rc_v[pl.ds(ch * L, L)]
            r_vec = lax.shift_right_logical(rc, 5)
            c_vec = lax.bitwise_and(rc, 31)
            score = None
            for k in range(K):
                wr = plsc.load_gather(w_v, [ksp[k], r_vec])
                wc = plsc.load_gather(w_v, [ksp[k], c_vec])
                prod = wr * wc
                score = prod if score is None else score + prod
            ec_cache = {}
            for j in range(L):
                pg = ch * L + j
                if pg >= P:
                    break
                r, c, po = _RS[pg], _CS[pg], _POS[pg] * D
                sv = jnp.zeros((L,), jnp.float32) + score[j]
                for h in (0, L):
                    if (c, h) not in ec_cache:
                        ec_cache[(c, h)] = e_v[pl.ds(c * D + h, L)]
                    er = e_v[pl.ds(r * D + h, L)]
                    obuf[pl.ds(po + h, L)] = er * ec_cache[(c, h)] * sv

        while state["issued"] < NBUF:
            issue_one()
        for ch in range(NCHUNK):
            need = _CMAX[ch] + 1
            while state["drained"] < need:
                drain_one()
                if state["issued"] < B:
                    issue_one()
            chunk_static(ch)

        pltpu.async_copy(obuf, out_hbm.at[pl.ds(fid * (P * D), P * D)],
                         osem).wait()


@jax.jit
def _ffm_sc(x, tab_t, w_t, rc):
    mesh = plsc.VectorSubcoreMesh(core_axis_name="c", subcore_axis_name="s")
    return pl.kernel(
        _body,
        out_type=jax.ShapeDtypeStruct((F * P * D,), jnp.float32),
        mesh=mesh,
        compiler_params=pltpu.CompilerParams(needs_layout_passes=False),
        scratch_types=[
            pltpu.VMEM((B, F), jnp.int32),
            pltpu.VMEM((K, B), jnp.float32),
            pltpu.VMEM((PPAD,), jnp.int32),
            pltpu.VMEM((B * D,), jnp.float32),
            pltpu.VMEM((P * D,), jnp.float32),
            [pltpu.VMEM((D, 128), jnp.float32) for _ in range(NBUF)],
            [pltpu.SemaphoreType.DMA for _ in range(NBUF)],
            pltpu.SemaphoreType.DMA,
        ],
    )(x, tab_t, w_t, rc)


def kernel(x, tables, W):
    tab_t = jnp.swapaxes(tables, 1, 2)
    w_t = jnp.swapaxes(W, 0, 1)
    rc = jnp.asarray(_RC_NP)
    return _ffm_sc(x, tab_t, w_t, rc).reshape(F, P * D)
